# baseline (speedup 1.0000x reference)
import functools

import jax
import jax.numpy as jnp
from jax import lax
from jax.experimental import pallas as pl
from jax.experimental.pallas import tpu as pltpu
from jax.experimental.pallas import tpu_sc as plsc


_PATCH = 8


@functools.lru_cache(maxsize=None)
def _make_sc_gather(num_idx: int, dim: int, vocab: int):
    info = plsc.get_sparse_core_info()
    nc, ns, nl = info.num_cores, info.num_subcores, info.num_lanes
    nw = nc * ns
    rows_per_w = num_idx // nw
    groups = rows_per_w // nl
    words_per_w = rows_per_w * dim
    n_wr = 4
    mesh = plsc.VectorSubcoreMesh(core_axis_name="c", subcore_axis_name="s")

    @functools.partial(
        pl.kernel,
        mesh=mesh,
        out_type=jax.ShapeDtypeStruct((num_idx * dim,), jnp.int32),
        scratch_types=[
            pltpu.VMEM((vocab * dim,), jnp.int32),
            pltpu.VMEM((rows_per_w,), jnp.int32),
            pltpu.VMEM((words_per_w,), jnp.int32),
            pltpu.SemaphoreType.DMA,
        ],
        compiler_params=pltpu.CompilerParams(
            use_tc_tiling_on_sc=False, needs_layout_passes=False
        ),
    )
    def gather(idx_hbm, table_hbm, out_hbm, table_v, idx_v, rows_v, sem_w):
        wid = lax.axis_index("s") * nc + lax.axis_index("c")
        base = wid * rows_per_w
        pltpu.sync_copy(idx_hbm.at[pl.ds(base, rows_per_w)], idx_v)
        pltpu.sync_copy(table_hbm, table_v)

        lane = lax.iota(jnp.int32, nl)
        lane_dim = lane * dim
        g_per_wr = groups // n_wr
        w_words = words_per_w // n_wr

        def _write_chunk(q):
            return pltpu.make_async_copy(
                rows_v.at[pl.ds(q * w_words, w_words)],
                out_hbm.at[pl.ds(wid * words_per_w + q * w_words, w_words)],
                sem_w,
            )

        for q in range(n_wr):

            @pl.loop(q * g_per_wr, (q + 1) * g_per_wr)
            def _gather_group(g):
                bytes16 = idx_v[pl.ds(g * nl, nl)]
                src = bytes16 * dim
                dst = lane_dim + g * (nl * dim)
                for j in range(dim):
                    vals = plsc.load_gather(table_v, [src + j])
                    plsc.store_scatter(rows_v, [dst + j], vals)

            _write_chunk(q).start()

        for q in range(n_wr):
            _write_chunk(q).wait()

    return gather


def _mm_body(m_ref, w0_ref, w1_ref, b_ref, o_ref):
    bm, kw = m_ref.shape
    xb = pltpu.bitcast(m_ref[...], jnp.bfloat16)
    x3 = xb.reshape(bm, 2, kw)
    a0 = x3[:, 0, :]
    a1 = x3[:, 1, :]
    o_ref[...] = (
        jnp.dot(a0, w0_ref[0], preferred_element_type=jnp.float32)
        + jnp.dot(a1, w1_ref[0], preferred_element_type=jnp.float32)
        + b_ref[...][None, :]
    )


def _tc_matmul(m2d, w2, b, bm):
    m, kw = m2d.shape
    n = w2.shape[2]
    return pl.pallas_call(
        _mm_body,
        grid=(m // bm,),
        in_specs=[
            pl.BlockSpec((bm, kw), lambda i: (i, 0)),
            pl.BlockSpec((1, kw, n), lambda i: (0, 0, 0)),
            pl.BlockSpec((1, kw, n), lambda i: (1, 0, 0)),
            pl.BlockSpec((n,), lambda i: (0,)),
        ],
        out_specs=pl.BlockSpec((bm, n), lambda i: (i, 0)),
        out_shape=jax.ShapeDtypeStruct((m, n), jnp.float32),
        compiler_params=pltpu.CompilerParams(
            dimension_semantics=("arbitrary",),
        ),
    )(m2d, w2, w2, b)


def kernel(bytes_flat, table, W, b):
    B, L = bytes_flat.shape
    P = _PATCH
    T = L // P
    byte_dim = table.shape[1]
    n_idx = B * T * P
    dim_w = byte_dim // 2

    idx1d = bytes_flat[:, : T * P].reshape(n_idx)
    table_pk = lax.bitcast_convert_type(
        table.astype(jnp.bfloat16).reshape(table.shape[0], dim_w, 2),
        jnp.int32,
    ).reshape(table.shape[0] * dim_w)
    gather = _make_sc_gather(n_idx, dim_w, table.shape[0])
    embs = gather(idx1d, table_pk)

    m2d = embs.reshape(B * T, P * dim_w)
    w_bf = W.astype(jnp.bfloat16)
    w2 = jnp.stack([w_bf[0::2], w_bf[1::2]])
    out = _tc_matmul(m2d, w2, b, 1024)
    return out.reshape(B, T, -1), T


# speedup vs baseline: 1.1621x; 1.1621x over previous
import functools

import jax
import jax.numpy as jnp
from jax import lax
from jax.experimental import pallas as pl
from jax.experimental.pallas import tpu as pltpu
from jax.experimental.pallas import tpu_sc as plsc


_PATCH = 8


@functools.lru_cache(maxsize=None)
def _make_sc_gather(num_idx: int, dim: int, vocab: int):
    info = plsc.get_sparse_core_info()
    nc, ns, nl = info.num_cores, info.num_subcores, info.num_lanes
    nw = nc * ns
    rows_per_w = num_idx // nw
    groups = rows_per_w // nl
    words_per_w = rows_per_w * dim
    n_wr = 4
    mesh = plsc.VectorSubcoreMesh(core_axis_name="c", subcore_axis_name="s")

    @functools.partial(
        pl.kernel,
        mesh=mesh,
        out_type=jax.ShapeDtypeStruct((num_idx * dim,), jnp.int32),
        scratch_types=[
            pltpu.VMEM((vocab * dim,), jnp.int32),
            pltpu.VMEM((rows_per_w,), jnp.int32),
            pltpu.VMEM((words_per_w,), jnp.int32),
            pltpu.SemaphoreType.DMA,
        ],
        compiler_params=pltpu.CompilerParams(
            use_tc_tiling_on_sc=False, needs_layout_passes=False
        ),
    )
    def gather(idx_hbm, table_hbm, out_hbm, table_v, idx_v, rows_v, sem_w):
        wid = lax.axis_index("s") * nc + lax.axis_index("c")
        base = wid * rows_per_w
        pltpu.sync_copy(idx_hbm.at[pl.ds(base, rows_per_w)], idx_v)
        pltpu.sync_copy(table_hbm, table_v)

        lane = lax.iota(jnp.int32, nl)
        lane_dim = lane * dim
        g_per_wr = groups // n_wr
        w_words = words_per_w // n_wr

        def _write_chunk(q):
            return pltpu.make_async_copy(
                rows_v.at[pl.ds(q * w_words, w_words)],
                out_hbm.at[pl.ds(wid * words_per_w + q * w_words, w_words)],
                sem_w,
            )

        for q in range(n_wr):

            @pl.loop(q * g_per_wr, (q + 1) * g_per_wr)
            def _gather_group(g):
                bytes16 = idx_v[pl.ds(g * nl, nl)]
                dst = lane_dim + g * (nl * dim)
                for j in range(dim):
                    vals = plsc.load_gather(table_v, [bytes16 + j * vocab])
                    plsc.store_scatter(rows_v, [dst + j], vals)

            _write_chunk(q).start()

        for q in range(n_wr):
            _write_chunk(q).wait()

    return gather


def _mm_body(m_ref, w0_ref, w1_ref, b_ref, o_ref):
    bm, kw = m_ref.shape
    xb = pltpu.bitcast(m_ref[...], jnp.bfloat16)
    x3 = xb.reshape(bm, 2, kw)
    a0 = x3[:, 0, :]
    a1 = x3[:, 1, :]
    o_ref[...] = (
        jnp.dot(a0, w0_ref[0], preferred_element_type=jnp.float32)
        + jnp.dot(a1, w1_ref[0], preferred_element_type=jnp.float32)
        + b_ref[...][None, :]
    )


def _tc_matmul(m2d, w2, b, bm):
    m, kw = m2d.shape
    n = w2.shape[2]
    return pl.pallas_call(
        _mm_body,
        grid=(m // bm,),
        in_specs=[
            pl.BlockSpec((bm, kw), lambda i: (i, 0)),
            pl.BlockSpec((1, kw, n), lambda i: (0, 0, 0)),
            pl.BlockSpec((1, kw, n), lambda i: (1, 0, 0)),
            pl.BlockSpec((n,), lambda i: (0,)),
        ],
        out_specs=pl.BlockSpec((bm, n), lambda i: (i, 0)),
        out_shape=jax.ShapeDtypeStruct((m, n), jnp.float32),
        compiler_params=pltpu.CompilerParams(
            dimension_semantics=("arbitrary",),
        ),
    )(m2d, w2, w2, b)


def kernel(bytes_flat, table, W, b):
    B, L = bytes_flat.shape
    P = _PATCH
    T = L // P
    byte_dim = table.shape[1]
    n_idx = B * T * P
    dim_w = byte_dim // 2

    idx1d = bytes_flat[:, : T * P].reshape(n_idx)
    table_pk = (
        lax.bitcast_convert_type(
            table.astype(jnp.bfloat16).reshape(table.shape[0], dim_w, 2),
            jnp.int32,
        )
        .T.reshape(table.shape[0] * dim_w)
    )
    gather = _make_sc_gather(n_idx, dim_w, table.shape[0])
    embs = gather(idx1d, table_pk)

    m2d = embs.reshape(B * T, P * dim_w)
    w_bf = W.astype(jnp.bfloat16)
    w2 = jnp.stack([w_bf[0::2], w_bf[1::2]])
    out = _tc_matmul(m2d, w2, b, 1024)
    return out.reshape(B, T, -1), T


# speedup vs baseline: 1.1900x; 1.0239x over previous
import functools

import jax
import jax.numpy as jnp
from jax import lax
from jax.experimental import pallas as pl
from jax.experimental.pallas import tpu as pltpu
from jax.experimental.pallas import tpu_sc as plsc


_PATCH = 8


@functools.lru_cache(maxsize=None)
def _make_sc_gather(num_idx: int, dim: int, vocab: int):
    info = plsc.get_sparse_core_info()
    nc, ns, nl = info.num_cores, info.num_subcores, info.num_lanes
    nw = nc * ns
    rows_per_w = num_idx // nw
    groups = rows_per_w // nl
    words_per_w = rows_per_w * dim
    n_wr = 4
    mesh = plsc.VectorSubcoreMesh(core_axis_name="c", subcore_axis_name="s")

    @functools.partial(
        pl.kernel,
        mesh=mesh,
        out_type=jax.ShapeDtypeStruct((num_idx * dim,), jnp.int32),
        scratch_types=[
            pltpu.VMEM((vocab * dim,), jnp.int32),
            pltpu.VMEM((rows_per_w,), jnp.int32),
            pltpu.VMEM((words_per_w,), jnp.int32),
            pltpu.SemaphoreType.DMA,
        ],
        compiler_params=pltpu.CompilerParams(
            use_tc_tiling_on_sc=False, needs_layout_passes=False
        ),
    )
    def gather(idx_hbm, table_hbm, out_hbm, table_v, idx_v, rows_v, sem_w):
        wid = lax.axis_index("s") * nc + lax.axis_index("c")
        base = wid * rows_per_w
        pltpu.sync_copy(idx_hbm.at[pl.ds(base, rows_per_w)], idx_v)
        pltpu.sync_copy(table_hbm, table_v)

        lane = lax.iota(jnp.int32, nl)
        g_per_wr = groups // n_wr
        w_words = words_per_w // n_wr

        def _write_chunk(q):
            return pltpu.make_async_copy(
                rows_v.at[pl.ds(q * w_words, w_words)],
                out_hbm.at[pl.ds(wid * words_per_w + q * w_words, w_words)],
                sem_w,
            )

        for q in range(n_wr):

            @pl.loop(q * g_per_wr, (q + 1) * g_per_wr, unroll=2)
            def _gather_group(g):
                bytes16 = idx_v[pl.ds(g * nl, nl)]
                base = g * (nl * dim)
                for i in range(nl):
                    byte = bytes16[i]
                    vals = plsc.load_gather(table_v, [byte * dim + lane])
                    rows_v[pl.ds(base + i * dim, dim)] = vals

            _write_chunk(q).start()

        for q in range(n_wr):
            _write_chunk(q).wait()

    return gather


def _mm_body(m_ref, w0_ref, w1_ref, b_ref, o_ref):
    bm, kw = m_ref.shape
    xb = pltpu.bitcast(m_ref[...], jnp.bfloat16)
    x3 = xb.reshape(bm, 2, kw)
    a0 = x3[:, 0, :]
    a1 = x3[:, 1, :]
    o_ref[...] = (
        jnp.dot(a0, w0_ref[0], preferred_element_type=jnp.float32)
        + jnp.dot(a1, w1_ref[0], preferred_element_type=jnp.float32)
        + b_ref[...][None, :]
    )


def _tc_matmul(m2d, w2, b, bm):
    m, kw = m2d.shape
    n = w2.shape[2]
    return pl.pallas_call(
        _mm_body,
        grid=(m // bm,),
        in_specs=[
            pl.BlockSpec((bm, kw), lambda i: (i, 0)),
            pl.BlockSpec((1, kw, n), lambda i: (0, 0, 0)),
            pl.BlockSpec((1, kw, n), lambda i: (1, 0, 0)),
            pl.BlockSpec((n,), lambda i: (0,)),
        ],
        out_specs=pl.BlockSpec((bm, n), lambda i: (i, 0)),
        out_shape=jax.ShapeDtypeStruct((m, n), jnp.float32),
        compiler_params=pltpu.CompilerParams(
            dimension_semantics=("arbitrary",),
        ),
    )(m2d, w2, w2, b)


def kernel(bytes_flat, table, W, b):
    B, L = bytes_flat.shape
    P = _PATCH
    T = L // P
    byte_dim = table.shape[1]
    n_idx = B * T * P
    dim_w = byte_dim // 2

    idx1d = bytes_flat[:, : T * P].reshape(n_idx)
    table_pk = lax.bitcast_convert_type(
        table.astype(jnp.bfloat16).reshape(table.shape[0], dim_w, 2),
        jnp.int32,
    ).reshape(table.shape[0] * dim_w)
    gather = _make_sc_gather(n_idx, dim_w, table.shape[0])
    embs = gather(idx1d, table_pk)

    m2d = embs.reshape(B * T, P * dim_w)
    w_bf = W.astype(jnp.bfloat16)
    w2 = jnp.stack([w_bf[0::2], w_bf[1::2]])
    out = _tc_matmul(m2d, w2, b, 1024)
    return out.reshape(B, T, -1), T


# speedup vs baseline: 1.2213x; 1.0263x over previous
import functools

import jax
import jax.numpy as jnp
from jax import lax
from jax.experimental import pallas as pl
from jax.experimental.pallas import tpu as pltpu
from jax.experimental.pallas import tpu_sc as plsc


_PATCH = 8


@functools.lru_cache(maxsize=None)
def _make_sc_gather(num_idx: int, dim: int, vocab: int):
    info = plsc.get_sparse_core_info()
    nc, ns, nl = info.num_cores, info.num_subcores, info.num_lanes
    nw = nc * ns
    rows_per_w = num_idx // nw
    groups = rows_per_w // nl
    words_per_w = rows_per_w * dim
    n_wr = 4
    mesh = plsc.VectorSubcoreMesh(core_axis_name="c", subcore_axis_name="s")

    stream_rows = rows_per_w // 2
    stream_chunks = stream_rows // 128

    @functools.partial(
        pl.kernel,
        mesh=mesh,
        out_type=jax.ShapeDtypeStruct((num_idx, dim), jnp.int32),
        scratch_types=[
            pltpu.VMEM((vocab, dim), jnp.int32),
            pltpu.VMEM((rows_per_w,), jnp.int32),
            pltpu.VMEM((rows_per_w, dim), jnp.int32),
            pltpu.SemaphoreType.DMA,
            pltpu.SemaphoreType.DMA,
        ],
        compiler_params=pltpu.CompilerParams(
            use_tc_tiling_on_sc=False, needs_layout_passes=False
        ),
    )
    def gather(
        idx_hbm, table_hbm, out_hbm,
        table_v, idx_v, rows_v, sem_g, sem_w,
    ):
        wid = lax.axis_index("s") * nc + lax.axis_index("c")
        base = wid * rows_per_w
        pltpu.sync_copy(idx_hbm.at[pl.ds(base, rows_per_w)], idx_v)

        def _stream_chunk(ci):
            return pltpu.make_async_copy(
                table_hbm.at[idx_v.at[pl.ds(ci * 128, 128)]],
                rows_v.at[pl.ds(ci * 128, 128)],
                sem_g,
            )

        for ci in range(stream_chunks):
            _stream_chunk(ci).start()

        pltpu.sync_copy(table_hbm, table_v)

        lane = lax.iota(jnp.int32, nl)
        g_per_wr = groups // n_wr
        r_per_wr = rows_per_w // n_wr

        def _write_chunk(q):
            return pltpu.make_async_copy(
                rows_v.at[pl.ds(q * r_per_wr, r_per_wr)],
                out_hbm.at[pl.ds(base + q * r_per_wr, r_per_wr)],
                sem_w,
            )

        half_wr = n_wr // 2
        for q in range(half_wr, n_wr):

            @pl.loop(q * g_per_wr, (q + 1) * g_per_wr, unroll=2)
            def _gather_group(g):
                bytes16 = idx_v[pl.ds(g * nl, nl)]
                for i in range(nl):
                    byte = bytes16[i]
                    row16 = jnp.full((nl,), byte, dtype=jnp.int32)
                    vals = plsc.load_gather(table_v, [row16, lane])
                    rows_v[g * nl + i, :] = vals

            _write_chunk(q).start()

        for ci in range(stream_chunks):
            _stream_chunk(ci).wait()
        for q in range(half_wr):
            _write_chunk(q).start()

        for q in range(n_wr):
            _write_chunk(q).wait()

    return gather


def _mm_body(m_ref, w0_ref, w1_ref, b_ref, o_ref):
    bm, kw = m_ref.shape
    xb = pltpu.bitcast(m_ref[...], jnp.bfloat16)
    x3 = xb.reshape(bm, 2, kw)
    a0 = x3[:, 0, :]
    a1 = x3[:, 1, :]
    o_ref[...] = (
        jnp.dot(a0, w0_ref[0], preferred_element_type=jnp.float32)
        + jnp.dot(a1, w1_ref[0], preferred_element_type=jnp.float32)
        + b_ref[...][None, :]
    )


def _tc_matmul(m2d, w2, b, bm):
    m, kw = m2d.shape
    n = w2.shape[2]
    return pl.pallas_call(
        _mm_body,
        grid=(m // bm,),
        in_specs=[
            pl.BlockSpec((bm, kw), lambda i: (i, 0)),
            pl.BlockSpec((1, kw, n), lambda i: (0, 0, 0)),
            pl.BlockSpec((1, kw, n), lambda i: (1, 0, 0)),
            pl.BlockSpec((n,), lambda i: (0,)),
        ],
        out_specs=pl.BlockSpec((bm, n), lambda i: (i, 0)),
        out_shape=jax.ShapeDtypeStruct((m, n), jnp.float32),
        compiler_params=pltpu.CompilerParams(
            dimension_semantics=("arbitrary",),
        ),
    )(m2d, w2, w2, b)


def kernel(bytes_flat, table, W, b):
    B, L = bytes_flat.shape
    P = _PATCH
    T = L // P
    byte_dim = table.shape[1]
    n_idx = B * T * P
    dim_w = byte_dim // 2

    idx1d = bytes_flat[:, : T * P].reshape(n_idx)
    table_pk2 = lax.bitcast_convert_type(
        table.astype(jnp.bfloat16).reshape(table.shape[0], dim_w, 2),
        jnp.int32,
    )
    gather = _make_sc_gather(n_idx, dim_w, table.shape[0])
    embs = gather(idx1d, table_pk2)

    m2d = embs.reshape(B * T, P * dim_w)
    w_bf = W.astype(jnp.bfloat16)
    w2 = jnp.stack([w_bf[0::2], w_bf[1::2]])
    out = _tc_matmul(m2d, w2, b, 1024)
    return out.reshape(B, T, -1), T


# speedup vs baseline: 1.2346x; 1.0109x over previous
import functools

import jax
import jax.numpy as jnp
from jax import lax
from jax.experimental import pallas as pl
from jax.experimental.pallas import tpu as pltpu
from jax.experimental.pallas import tpu_sc as plsc


_PATCH = 8


@functools.lru_cache(maxsize=None)
def _make_sc_gather(num_idx: int, dim: int, vocab: int):
    info = plsc.get_sparse_core_info()
    nc, ns, nl = info.num_cores, info.num_subcores, info.num_lanes
    nw = nc * ns
    rows_per_w = num_idx // nw
    groups = rows_per_w // nl
    words_per_w = rows_per_w * dim
    n_wr = 4
    mesh = plsc.VectorSubcoreMesh(core_axis_name="c", subcore_axis_name="s")

    stream_rows = rows_per_w // 2
    stream_chunks = stream_rows // 128

    @functools.partial(
        pl.kernel,
        mesh=mesh,
        out_type=jax.ShapeDtypeStruct((num_idx, dim), jnp.int32),
        scratch_types=[
            pltpu.VMEM((vocab * dim,), jnp.int32),
            pltpu.VMEM((rows_per_w,), jnp.int32),
            pltpu.VMEM((rows_per_w, dim), jnp.int32),
            pltpu.SemaphoreType.DMA,
            pltpu.SemaphoreType.DMA,
        ],
        compiler_params=pltpu.CompilerParams(
            use_tc_tiling_on_sc=False, needs_layout_passes=False
        ),
    )
    def gather(
        idx_hbm, table_hbm, tableflat_hbm, out_hbm,
        table_v, idx_v, rows_v, sem_g, sem_w,
    ):
        wid = lax.axis_index("s") * nc + lax.axis_index("c")
        base = wid * rows_per_w
        pltpu.sync_copy(idx_hbm.at[pl.ds(base, rows_per_w)], idx_v)

        def _stream_chunk(ci):
            return pltpu.make_async_copy(
                table_hbm.at[idx_v.at[pl.ds(ci * 128, 128)]],
                rows_v.at[pl.ds(ci * 128, 128)],
                sem_g,
            )

        for ci in range(stream_chunks):
            _stream_chunk(ci).start()

        pltpu.sync_copy(tableflat_hbm, table_v)

        lane = lax.iota(jnp.int32, nl)
        g_per_wr = groups // n_wr
        r_per_wr = rows_per_w // n_wr

        def _write_chunk(q):
            return pltpu.make_async_copy(
                rows_v.at[pl.ds(q * r_per_wr, r_per_wr)],
                out_hbm.at[pl.ds(base + q * r_per_wr, r_per_wr)],
                sem_w,
            )

        half_wr = n_wr // 2
        for q in range(half_wr, n_wr):

            @pl.loop(q * g_per_wr, (q + 1) * g_per_wr, unroll=2)
            def _gather_group(g):
                bytes16 = idx_v[pl.ds(g * nl, nl)]
                for i in range(nl):
                    byte = bytes16[i]
                    vals = plsc.load_gather(table_v, [byte * dim + lane])
                    rows_v[g * nl + i, :] = vals

            _write_chunk(q).start()

        for ci in range(stream_chunks):
            _stream_chunk(ci).wait()
        for q in range(half_wr):
            _write_chunk(q).start()

        for q in range(n_wr):
            _write_chunk(q).wait()

    return gather


def _mm_body(m_ref, w0_ref, w1_ref, b_ref, o_ref):
    bm, kw = m_ref.shape
    xb = pltpu.bitcast(m_ref[...], jnp.bfloat16)
    x3 = xb.reshape(bm, 2, kw)
    a0 = x3[:, 0, :]
    a1 = x3[:, 1, :]
    o_ref[...] = (
        jnp.dot(a0, w0_ref[0], preferred_element_type=jnp.float32)
        + jnp.dot(a1, w1_ref[0], preferred_element_type=jnp.float32)
        + b_ref[...][None, :]
    )


def _tc_matmul(m2d, w2, b, bm):
    m, kw = m2d.shape
    n = w2.shape[2]
    return pl.pallas_call(
        _mm_body,
        grid=(m // bm,),
        in_specs=[
            pl.BlockSpec((bm, kw), lambda i: (i, 0)),
            pl.BlockSpec((1, kw, n), lambda i: (0, 0, 0)),
            pl.BlockSpec((1, kw, n), lambda i: (1, 0, 0)),
            pl.BlockSpec((n,), lambda i: (0,)),
        ],
        out_specs=pl.BlockSpec((bm, n), lambda i: (i, 0)),
        out_shape=jax.ShapeDtypeStruct((m, n), jnp.float32),
        compiler_params=pltpu.CompilerParams(
            dimension_semantics=("arbitrary",),
        ),
    )(m2d, w2, w2, b)


def kernel(bytes_flat, table, W, b):
    B, L = bytes_flat.shape
    P = _PATCH
    T = L // P
    byte_dim = table.shape[1]
    n_idx = B * T * P
    dim_w = byte_dim // 2

    idx1d = bytes_flat[:, : T * P].reshape(n_idx)
    table_pk2 = lax.bitcast_convert_type(
        table.astype(jnp.bfloat16).reshape(table.shape[0], dim_w, 2),
        jnp.int32,
    )
    table_pk1 = lax.bitcast_convert_type(
        table.astype(jnp.bfloat16).reshape(table.shape[0] * dim_w, 2),
        jnp.int32,
    )
    gather = _make_sc_gather(n_idx, dim_w, table.shape[0])
    embs = gather(idx1d, table_pk2, table_pk1)

    m2d = embs.reshape(B * T, P * dim_w)
    w_bf = W.astype(jnp.bfloat16)
    w2 = jnp.stack([w_bf[0::2], w_bf[1::2]])
    out = _tc_matmul(m2d, w2, b, 1024)
    return out.reshape(B, T, -1), T


# speedup vs baseline: 1.2420x; 1.0060x over previous
import functools

import jax
import jax.numpy as jnp
from jax import lax
from jax.experimental import pallas as pl
from jax.experimental.pallas import tpu as pltpu
from jax.experimental.pallas import tpu_sc as plsc


_PATCH = 8


@functools.lru_cache(maxsize=None)
def _make_sc_gather(num_idx: int, dim: int, vocab: int):
    info = plsc.get_sparse_core_info()
    nc, ns, nl = info.num_cores, info.num_subcores, info.num_lanes
    nw = nc * ns
    rows_per_w = num_idx // nw
    groups = rows_per_w // nl
    words_per_w = rows_per_w * dim
    n_wr = 4
    mesh = plsc.VectorSubcoreMesh(core_axis_name="c", subcore_axis_name="s")

    stream_rows = rows_per_w // 2
    stream_chunks = stream_rows // 128

    @functools.partial(
        pl.kernel,
        mesh=mesh,
        out_type=jax.ShapeDtypeStruct((num_idx, dim), jnp.int32),
        scratch_types=[
            pltpu.VMEM((vocab * dim,), jnp.int32),
            pltpu.VMEM((rows_per_w,), jnp.int32),
            pltpu.VMEM((rows_per_w, dim), jnp.int32),
            pltpu.SemaphoreType.DMA,
            pltpu.SemaphoreType.DMA,
        ],
        compiler_params=pltpu.CompilerParams(
            use_tc_tiling_on_sc=False, needs_layout_passes=False
        ),
    )
    def gather(
        idx_hbm, table_hbm, tableflat_hbm, out_hbm,
        table_v, idx_v, rows_v, sem_g, sem_w,
    ):
        wid = lax.axis_index("s") * nc + lax.axis_index("c")
        base = wid * rows_per_w
        pltpu.sync_copy(idx_hbm.at[pl.ds(base, rows_per_w)], idx_v)

        def _stream_chunk(ci):
            return pltpu.make_async_copy(
                table_hbm.at[idx_v.at[pl.ds(ci * 128, 128)]],
                rows_v.at[pl.ds(ci * 128, 128)],
                sem_g,
            )

        for ci in range(stream_chunks):
            _stream_chunk(ci).start()

        pltpu.sync_copy(tableflat_hbm, table_v)

        lane = lax.iota(jnp.int32, nl)
        g_per_wr = groups // n_wr
        r_per_wr = rows_per_w // n_wr

        def _write_chunk(q):
            return pltpu.make_async_copy(
                rows_v.at[pl.ds(q * r_per_wr, r_per_wr)],
                out_hbm.at[pl.ds(base + q * r_per_wr, r_per_wr)],
                sem_w,
            )

        half_wr = n_wr // 2
        for q in range(half_wr, n_wr):

            @pl.loop(q * g_per_wr, (q + 1) * g_per_wr, unroll=2)
            def _gather_group(g):
                bytes16 = idx_v[pl.ds(g * nl, nl)]
                for i in range(nl):
                    byte = bytes16[i]
                    vals = plsc.load_gather(table_v, [byte * dim + lane])
                    rows_v[g * nl + i, :] = vals

            _write_chunk(q).start()

        for ci in range(stream_chunks):
            _stream_chunk(ci).wait()
        for q in range(half_wr):
            _write_chunk(q).start()

        for q in range(n_wr):
            _write_chunk(q).wait()

    return gather


def _mm_body(m_ref, w0_ref, w1_ref, b_ref, o_ref):
    bm, kw = m_ref.shape
    xb = pltpu.bitcast(m_ref[...], jnp.bfloat16)
    x3 = xb.reshape(bm, 2, kw)
    a0 = x3[:, 0, :]
    a1 = x3[:, 1, :]
    o_ref[...] = (
        jnp.dot(a0, w0_ref[0], preferred_element_type=jnp.float32)
        + jnp.dot(a1, w1_ref[0], preferred_element_type=jnp.float32)
        + b_ref[...][None, :]
    )


def _tc_matmul(m2d, w2, b, bm):
    m, kw = m2d.shape
    n = w2.shape[2]
    return pl.pallas_call(
        _mm_body,
        grid=(m // bm,),
        in_specs=[
            pl.BlockSpec((bm, kw), lambda i: (i, 0)),
            pl.BlockSpec((1, kw, n), lambda i: (0, 0, 0)),
            pl.BlockSpec((1, kw, n), lambda i: (1, 0, 0)),
            pl.BlockSpec((n,), lambda i: (0,)),
        ],
        out_specs=pl.BlockSpec((bm, n), lambda i: (i, 0)),
        out_shape=jax.ShapeDtypeStruct((m, n), jnp.float32),
        compiler_params=pltpu.CompilerParams(
            dimension_semantics=("arbitrary",),
        ),
    )(m2d, w2, w2, b)


def kernel(bytes_flat, table, W, b):
    B, L = bytes_flat.shape
    P = _PATCH
    T = L // P
    byte_dim = table.shape[1]
    n_idx = B * T * P
    dim_w = byte_dim // 2

    idx1d = bytes_flat[:, : T * P].reshape(n_idx)
    table_pk2 = lax.bitcast_convert_type(
        table.astype(jnp.bfloat16).reshape(table.shape[0], dim_w, 2),
        jnp.int32,
    )
    table_pk1 = lax.bitcast_convert_type(
        table.astype(jnp.bfloat16).reshape(table.shape[0] * dim_w, 2),
        jnp.int32,
    )
    gather = _make_sc_gather(n_idx, dim_w, table.shape[0])
    embs = gather(idx1d, table_pk2, table_pk1)

    m2d = embs.reshape(B * T, P * dim_w)
    w_bf = W.astype(jnp.bfloat16)
    w2 = jnp.stack([w_bf[0::2], w_bf[1::2]])
    out = _tc_matmul(m2d, w2, b, 2048)
    return out.reshape(B, T, -1), T
